# Initial kernel scaffold; baseline (speedup 1.0000x reference)
import jax
import jax.numpy as jnp
from jax.experimental import pallas as pl


def kernel(x, edge_index, W_in, b_in, W_out, b_out, W_root, b_root):
    raise NotImplementedError("write your pallas kernel here")


# speedup vs baseline: 2.6629x; 2.6629x over previous
import functools

import jax
import jax.numpy as jnp
from jax import lax
from jax.experimental import pallas as pl
from jax.experimental.pallas import tpu as pltpu
from jax.experimental.pallas import tpu_sc as plsc

N_NODES = 10000
N_EDGES = 160000
D = 256
ALPHA = 0.5

H = 128
NS = 16
EPT = N_EDGES // NS
B = 80
NB = EPT // B
N_PAD = 10240
SLAB = N_PAD // NS
NCH = SLAB // B
DEG_W = 16


def _sc_segment_sums(xr, gf, gb, src, dst, ds_flat, zeros_blk, ones_blk):
  mesh = plsc.VectorSubcoreMesh(core_axis_name="c", subcore_axis_name="s")
  out_type = [
      jax.ShapeDtypeStruct((2 * N_PAD, H), jnp.float32),
      jax.ShapeDtypeStruct((2 * N_PAD, H), jnp.float32),
      jax.ShapeDtypeStruct((2 * N_PAD, H), jnp.float32),
  ]

  @functools.partial(
      pl.kernel,
      out_type=out_type,
      mesh=mesh,
      scratch_types=[
          pltpu.VMEM_SHARED((N_PAD, H), jnp.float32),
          pltpu.VMEM((B,), jnp.int32),
          pltpu.VMEM((B,), jnp.int32),
          pltpu.VMEM((B, H), jnp.float32),
          pltpu.SemaphoreType.DMA,
      ],
  )
  def sc_kernel(xr_hbm, gf_hbm, gb_hbm, src_hbm, dst_hbm, ds_hbm, zf_hbm,
                ones_hbm, sf_big, sb_big, deg_big,
                acc, idx_g, idx_s, rows, sem):
    cid = lax.axis_index("c")
    sid = lax.axis_index("s")
    z0 = sid * SLAB
    e0 = sid * EPT
    g0 = cid * N_EDGES + e0
    o0 = cid * N_PAD + z0

    def zero_acc():
      for k in range(NCH):
        pltpu.sync_copy(zf_hbm, acc.at[pl.ds(z0 + k * B, B)])

    def drain_acc(out_big):
      for k in range(NCH):
        pltpu.sync_copy(acc.at[pl.ds(z0 + k * B, B)], rows)
        pltpu.sync_copy(rows, out_big.at[pl.ds(o0 + k * B, B)])
      plsc.subcore_barrier()

    def feat_pass(g_hbm, s_hbm, out_big):
      zero_acc()
      plsc.subcore_barrier()

      def body(j, carry):
        base = j * B
        pltpu.sync_copy(g_hbm.at[pl.ds(g0 + base, B)], idx_g)
        pltpu.sync_copy(s_hbm.at[pl.ds(e0 + base, B)], idx_s)
        pltpu.async_copy(xr_hbm.at[idx_g], rows, sem).wait()
        pltpu.sync_copy(rows, acc.at[idx_s], add=True)
        return carry

      lax.fori_loop(0, NB, body, 0)
      plsc.subcore_barrier()
      drain_acc(out_big)

    def deg_pass():
      zero_acc()
      pltpu.sync_copy(ones_hbm, rows)
      plsc.subcore_barrier()

      def body(j, carry):
        pltpu.sync_copy(ds_hbm.at[pl.ds(g0 + j * B, B)], idx_s)
        pltpu.sync_copy(rows, acc.at[idx_s], add=True)
        return carry

      lax.fori_loop(0, NB, body, 0)
      plsc.subcore_barrier()
      drain_acc(deg_big)

    feat_pass(gf_hbm, dst_hbm, sf_big)
    feat_pass(gb_hbm, src_hbm, sb_big)
    deg_pass()

  return sc_kernel(xr, gf, gb, src, dst, ds_flat, zeros_blk, ones_blk)


R = 400


def _tc_combine(sf0, sf1, sb0, sb1, df, db, x,
                wf0, wf1, wb0, wb1, wr, b_in, b_out, b_root):
  def body(sf0_r, sf1_r, sb0_r, sb1_r, df_r, db_r, x_r,
           wf0_r, wf1_r, wb0_r, wb1_r, wr_r, bi_r, bo_r, br_r, o_r):
    invf = 1.0 / jnp.maximum(df_r[...], 1.0)
    invb = 1.0 / jnp.maximum(db_r[...], 1.0)
    fwd = jnp.dot(sf0_r[...] * invf, wf0_r[...],
                  preferred_element_type=jnp.float32)
    fwd = fwd + jnp.dot(sf1_r[...] * invf, wf1_r[...],
                        preferred_element_type=jnp.float32)
    bwd = jnp.dot(sb0_r[...] * invb, wb0_r[...],
                  preferred_element_type=jnp.float32)
    bwd = bwd + jnp.dot(sb1_r[...] * invb, wb1_r[...],
                        preferred_element_type=jnp.float32)
    root = jnp.dot(x_r[...], wr_r[...], preferred_element_type=jnp.float32)
    bias = ALPHA * bi_r[...] + (1.0 - ALPHA) * bo_r[...] + br_r[...]
    o_r[...] = ALPHA * fwd + (1.0 - ALPHA) * bwd + root + bias

  half = pl.BlockSpec((R, H), lambda i: (i, 0))
  col = pl.BlockSpec((R, 1), lambda i: (i, 0))
  whole = lambda s: pl.BlockSpec(s, lambda i: (0, 0))
  return pl.pallas_call(
      body,
      grid=(N_NODES // R,),
      in_specs=[half, half, half, half, col, col,
                pl.BlockSpec((R, D), lambda i: (i, 0)),
                whole((H, D)), whole((H, D)), whole((H, D)), whole((H, D)),
                whole((D, D)), whole((1, D)), whole((1, D)), whole((1, D))],
      out_specs=pl.BlockSpec((R, D), lambda i: (i, 0)),
      out_shape=jax.ShapeDtypeStruct((N_NODES, D), jnp.float32),
  )(sf0, sf1, sb0, sb1, df, db, x,
    wf0, wf1, wb0, wb1, wr, b_in, b_out, b_root)


def kernel(x, edge_index, W_in, b_in, W_out, b_out, W_root, b_root):
  ei = edge_index.astype(jnp.int32)
  src, dst = ei[0], ei[1]
  xr = x.reshape(2 * N_NODES, H)
  gf = jnp.concatenate([2 * src, 2 * src + 1])
  gb = jnp.concatenate([2 * dst, 2 * dst + 1])
  ds_flat = jnp.concatenate([dst, src])
  zeros_blk = jnp.zeros((B, H), jnp.float32)
  ones_blk = jnp.ones((B, H), jnp.float32)

  sf_big, sb_big, deg_big = _sc_segment_sums(
      xr, gf, gb, src, dst, ds_flat, zeros_blk, ones_blk)

  sf0, sf1 = sf_big[:N_NODES], sf_big[N_PAD:N_PAD + N_NODES]
  sb0, sb1 = sb_big[:N_NODES], sb_big[N_PAD:N_PAD + N_NODES]
  df = deg_big[:N_NODES, :1]
  db = deg_big[N_PAD:N_PAD + N_NODES, :1]

  wf = W_in.T
  wb = W_out.T
  return _tc_combine(
      sf0, sf1, sb0, sb1, df, db, x,
      wf[:H], wf[H:], wb[:H], wb[H:], W_root.T,
      b_in.reshape(1, D), b_out.reshape(1, D), b_root.reshape(1, D))


# speedup vs baseline: 3.4811x; 1.3073x over previous
import functools

import jax
import jax.numpy as jnp
from jax import lax
from jax.experimental import pallas as pl
from jax.experimental.pallas import tpu as pltpu
from jax.experimental.pallas import tpu_sc as plsc

N_NODES = 10000
N_EDGES = 160000
D = 256
ALPHA = 0.5

H = 128
NS = 16
EPT = N_EDGES // NS
B = 80
NB = EPT // B
N_PAD = 10240
SLAB = N_PAD // NS
NCH = SLAB // B
DEG_W = 16


def _sc_segment_sums(xr, gf, gb, src, dst, ds_flat, zeros_blk, ones_blk):
  mesh = plsc.VectorSubcoreMesh(core_axis_name="c", subcore_axis_name="s")
  out_type = [
      jax.ShapeDtypeStruct((2 * N_PAD, H), jnp.float32),
      jax.ShapeDtypeStruct((2 * N_PAD, H), jnp.float32),
      jax.ShapeDtypeStruct((2 * N_PAD, H), jnp.float32),
  ]

  @functools.partial(
      pl.kernel,
      out_type=out_type,
      mesh=mesh,
      scratch_types=[
          pltpu.VMEM_SHARED((N_PAD, H), jnp.float32),
          pltpu.VMEM((B,), jnp.int32),
          pltpu.VMEM((B,), jnp.int32),
          pltpu.VMEM((B,), jnp.int32),
          pltpu.VMEM((B,), jnp.int32),
          pltpu.VMEM((B, H), jnp.float32),
          pltpu.VMEM((B, H), jnp.float32),
          pltpu.SemaphoreType.DMA,
          pltpu.SemaphoreType.DMA,
      ],
  )
  def sc_kernel(xr_hbm, gf_hbm, gb_hbm, src_hbm, dst_hbm, ds_hbm, zf_hbm,
                ones_hbm, sf_big, sb_big, deg_big,
                acc, idx_g0, idx_s0, idx_g1, idx_s1, rows0, rows1,
                sem0, sem1):
    cid = lax.axis_index("c")
    sid = lax.axis_index("s")
    z0 = sid * SLAB
    e0 = sid * EPT
    g0 = cid * N_EDGES + e0
    o0 = cid * N_PAD + z0

    def zero_acc():
      for k in range(NCH):
        pltpu.sync_copy(zf_hbm, acc.at[pl.ds(z0 + k * B, B)])

    def drain_acc(out_big):
      for k in range(NCH):
        pltpu.sync_copy(acc.at[pl.ds(z0 + k * B, B)], rows0)
        pltpu.sync_copy(rows0, out_big.at[pl.ds(o0 + k * B, B)])
      plsc.subcore_barrier()

    def feat_pass(g_hbm, s_hbm, out_big):
      zero_acc()
      plsc.subcore_barrier()

      def body2(t, carry):
        b0 = 2 * t * B
        b1 = b0 + B
        pltpu.sync_copy(g_hbm.at[pl.ds(g0 + b0, B)], idx_g0)
        pltpu.sync_copy(s_hbm.at[pl.ds(e0 + b0, B)], idx_s0)
        cp0 = pltpu.async_copy(xr_hbm.at[idx_g0], rows0, sem0)
        pltpu.sync_copy(g_hbm.at[pl.ds(g0 + b1, B)], idx_g1)
        pltpu.sync_copy(s_hbm.at[pl.ds(e0 + b1, B)], idx_s1)
        cp1 = pltpu.async_copy(xr_hbm.at[idx_g1], rows1, sem1)
        cp0.wait()
        pltpu.sync_copy(rows0, acc.at[idx_s0], add=True)
        cp1.wait()
        pltpu.sync_copy(rows1, acc.at[idx_s1], add=True)
        return carry

      lax.fori_loop(0, NB // 2, body2, 0)
      bt = (NB - 1) * B
      pltpu.sync_copy(g_hbm.at[pl.ds(g0 + bt, B)], idx_g0)
      pltpu.sync_copy(s_hbm.at[pl.ds(e0 + bt, B)], idx_s0)
      pltpu.async_copy(xr_hbm.at[idx_g0], rows0, sem0).wait()
      pltpu.sync_copy(rows0, acc.at[idx_s0], add=True)
      plsc.subcore_barrier()
      drain_acc(out_big)

    def deg_pass():
      zero_acc()
      pltpu.sync_copy(ones_hbm, rows0)
      plsc.subcore_barrier()

      def body2(t, carry):
        b0 = 2 * t * B
        b1 = b0 + B
        pltpu.sync_copy(ds_hbm.at[pl.ds(g0 + b0, B)], idx_s0)
        cp0 = pltpu.async_copy(rows0, acc.at[idx_s0], sem0, add=True)
        pltpu.sync_copy(ds_hbm.at[pl.ds(g0 + b1, B)], idx_s1)
        cp1 = pltpu.async_copy(rows0, acc.at[idx_s1], sem1, add=True)
        cp0.wait()
        cp1.wait()
        return carry

      lax.fori_loop(0, NB // 2, body2, 0)
      bt = (NB - 1) * B
      pltpu.sync_copy(ds_hbm.at[pl.ds(g0 + bt, B)], idx_s0)
      pltpu.sync_copy(rows0, acc.at[idx_s0], add=True)
      plsc.subcore_barrier()
      drain_acc(deg_big)

    feat_pass(gf_hbm, dst_hbm, sf_big)
    feat_pass(gb_hbm, src_hbm, sb_big)
    deg_pass()

  return sc_kernel(xr, gf, gb, src, dst, ds_flat, zeros_blk, ones_blk)


R = 400


def _tc_combine(sf0, sf1, sb0, sb1, df, db, x,
                wf0, wf1, wb0, wb1, wr, b_in, b_out, b_root):
  def body(sf0_r, sf1_r, sb0_r, sb1_r, df_r, db_r, x_r,
           wf0_r, wf1_r, wb0_r, wb1_r, wr_r, bi_r, bo_r, br_r, o_r):
    invf = 1.0 / jnp.maximum(df_r[...], 1.0)
    invb = 1.0 / jnp.maximum(db_r[...], 1.0)
    fwd = jnp.dot(sf0_r[...] * invf, wf0_r[...],
                  preferred_element_type=jnp.float32)
    fwd = fwd + jnp.dot(sf1_r[...] * invf, wf1_r[...],
                        preferred_element_type=jnp.float32)
    bwd = jnp.dot(sb0_r[...] * invb, wb0_r[...],
                  preferred_element_type=jnp.float32)
    bwd = bwd + jnp.dot(sb1_r[...] * invb, wb1_r[...],
                        preferred_element_type=jnp.float32)
    root = jnp.dot(x_r[...], wr_r[...], preferred_element_type=jnp.float32)
    bias = ALPHA * bi_r[...] + (1.0 - ALPHA) * bo_r[...] + br_r[...]
    o_r[...] = ALPHA * fwd + (1.0 - ALPHA) * bwd + root + bias

  half = pl.BlockSpec((R, H), lambda i: (i, 0))
  col = pl.BlockSpec((R, 1), lambda i: (i, 0))
  whole = lambda s: pl.BlockSpec(s, lambda i: (0, 0))
  return pl.pallas_call(
      body,
      grid=(N_NODES // R,),
      in_specs=[half, half, half, half, col, col,
                pl.BlockSpec((R, D), lambda i: (i, 0)),
                whole((H, D)), whole((H, D)), whole((H, D)), whole((H, D)),
                whole((D, D)), whole((1, D)), whole((1, D)), whole((1, D))],
      out_specs=pl.BlockSpec((R, D), lambda i: (i, 0)),
      out_shape=jax.ShapeDtypeStruct((N_NODES, D), jnp.float32),
  )(sf0, sf1, sb0, sb1, df, db, x,
    wf0, wf1, wb0, wb1, wr, b_in, b_out, b_root)


def kernel(x, edge_index, W_in, b_in, W_out, b_out, W_root, b_root):
  ei = edge_index.astype(jnp.int32)
  src, dst = ei[0], ei[1]
  xr = x.reshape(2 * N_NODES, H)
  gf = jnp.concatenate([2 * src, 2 * src + 1])
  gb = jnp.concatenate([2 * dst, 2 * dst + 1])
  ds_flat = jnp.concatenate([dst, src])
  zeros_blk = jnp.zeros((B, H), jnp.float32)
  ones_blk = jnp.ones((B, H), jnp.float32)

  sf_big, sb_big, deg_big = _sc_segment_sums(
      xr, gf, gb, src, dst, ds_flat, zeros_blk, ones_blk)

  sf0, sf1 = sf_big[:N_NODES], sf_big[N_PAD:N_PAD + N_NODES]
  sb0, sb1 = sb_big[:N_NODES], sb_big[N_PAD:N_PAD + N_NODES]
  df = deg_big[:N_NODES, :1]
  db = deg_big[N_PAD:N_PAD + N_NODES, :1]

  wf = W_in.T
  wb = W_out.T
  return _tc_combine(
      sf0, sf1, sb0, sb1, df, db, x,
      wf[:H], wf[H:], wb[:H], wb[H:], W_root.T,
      b_in.reshape(1, D), b_out.reshape(1, D), b_root.reshape(1, D))


# speedup vs baseline: 3.4819x; 1.0002x over previous
import functools

import jax
import jax.numpy as jnp
from jax import lax
from jax.experimental import pallas as pl
from jax.experimental.pallas import tpu as pltpu
from jax.experimental.pallas import tpu_sc as plsc

N_NODES = 10000
N_EDGES = 160000
D = 256
ALPHA = 0.5

H = 128
NS = 16
EPT = N_EDGES // NS
B = 80
NB = EPT // B
N_PAD = 10240
SLAB = N_PAD // NS
NCH = SLAB // B
DEG_W = 16


def _sc_segment_sums(xr, gf, gb, src, dst, ds_flat, zeros_blk, ones_blk):
  mesh = plsc.VectorSubcoreMesh(core_axis_name="c", subcore_axis_name="s")
  out_type = [
      jax.ShapeDtypeStruct((2 * N_PAD, H), jnp.float32),
      jax.ShapeDtypeStruct((2 * N_PAD, H), jnp.float32),
      jax.ShapeDtypeStruct((2 * N_PAD, H), jnp.float32),
  ]

  @functools.partial(
      pl.kernel,
      out_type=out_type,
      mesh=mesh,
      scratch_types=[
          pltpu.VMEM_SHARED((N_PAD, H), jnp.float32),
          pltpu.VMEM((B,), jnp.int32),
          pltpu.VMEM((B,), jnp.int32),
          pltpu.VMEM((B,), jnp.int32),
          pltpu.VMEM((B,), jnp.int32),
          pltpu.VMEM((B, H), jnp.float32),
          pltpu.VMEM((B, H), jnp.float32),
          pltpu.SemaphoreType.DMA,
          pltpu.SemaphoreType.DMA,
          pltpu.SemaphoreType.DMA,
          pltpu.SemaphoreType.DMA,
      ],
  )
  def sc_kernel(xr_hbm, gf_hbm, gb_hbm, src_hbm, dst_hbm, ds_hbm, zf_hbm,
                ones_hbm, sf_big, sb_big, deg_big,
                acc, idx_g0, idx_s0, idx_g1, idx_s1, rows0, rows1,
                sem0, sem1, sem2, sem3):
    cid = lax.axis_index("c")
    sid = lax.axis_index("s")
    z0 = sid * SLAB
    e0 = sid * EPT
    g0 = cid * N_EDGES + e0
    o0 = cid * N_PAD + z0

    def zero_acc():
      for k in range(NCH):
        pltpu.sync_copy(zf_hbm, acc.at[pl.ds(z0 + k * B, B)])

    def drain_acc(out_big):
      for k in range(NCH):
        pltpu.sync_copy(acc.at[pl.ds(z0 + k * B, B)], rows0)
        pltpu.sync_copy(rows0, out_big.at[pl.ds(o0 + k * B, B)])
      plsc.subcore_barrier()

    def feat_pass(g_hbm, s_hbm, out_big):
      zero_acc()
      plsc.subcore_barrier()

      def body2(t, carry):
        b0 = 2 * t * B
        b1 = b0 + B
        pltpu.sync_copy(g_hbm.at[pl.ds(g0 + b0, B)], idx_g0)
        pltpu.sync_copy(s_hbm.at[pl.ds(e0 + b0, B)], idx_s0)
        cp0 = pltpu.async_copy(xr_hbm.at[idx_g0], rows0, sem0)
        pltpu.sync_copy(g_hbm.at[pl.ds(g0 + b1, B)], idx_g1)
        pltpu.sync_copy(s_hbm.at[pl.ds(e0 + b1, B)], idx_s1)
        cp1 = pltpu.async_copy(xr_hbm.at[idx_g1], rows1, sem1)
        cp0.wait()
        sc0 = pltpu.async_copy(rows0, acc.at[idx_s0], sem2, add=True)
        cp1.wait()
        sc1 = pltpu.async_copy(rows1, acc.at[idx_s1], sem3, add=True)
        sc0.wait()
        sc1.wait()
        return carry

      lax.fori_loop(0, NB // 2, body2, 0)
      bt = (NB - 1) * B
      pltpu.sync_copy(g_hbm.at[pl.ds(g0 + bt, B)], idx_g0)
      pltpu.sync_copy(s_hbm.at[pl.ds(e0 + bt, B)], idx_s0)
      pltpu.async_copy(xr_hbm.at[idx_g0], rows0, sem0).wait()
      pltpu.sync_copy(rows0, acc.at[idx_s0], add=True)
      plsc.subcore_barrier()
      drain_acc(out_big)

    def deg_pass():
      zero_acc()
      pltpu.sync_copy(ones_hbm, rows0)
      plsc.subcore_barrier()

      def body2(t, carry):
        b0 = 2 * t * B
        b1 = b0 + B
        pltpu.sync_copy(ds_hbm.at[pl.ds(g0 + b0, B)], idx_s0)
        cp0 = pltpu.async_copy(rows0, acc.at[idx_s0], sem0, add=True)
        pltpu.sync_copy(ds_hbm.at[pl.ds(g0 + b1, B)], idx_s1)
        cp1 = pltpu.async_copy(rows0, acc.at[idx_s1], sem1, add=True)
        cp0.wait()
        cp1.wait()
        return carry

      lax.fori_loop(0, NB // 2, body2, 0)
      bt = (NB - 1) * B
      pltpu.sync_copy(ds_hbm.at[pl.ds(g0 + bt, B)], idx_s0)
      pltpu.sync_copy(rows0, acc.at[idx_s0], add=True)
      plsc.subcore_barrier()
      drain_acc(deg_big)

    feat_pass(gf_hbm, dst_hbm, sf_big)
    feat_pass(gb_hbm, src_hbm, sb_big)
    deg_pass()

  return sc_kernel(xr, gf, gb, src, dst, ds_flat, zeros_blk, ones_blk)


R = 400


def _tc_combine(sf0, sf1, sb0, sb1, df, db, x,
                wf0, wf1, wb0, wb1, wr, b_in, b_out, b_root):
  def body(sf0_r, sf1_r, sb0_r, sb1_r, df_r, db_r, x_r,
           wf0_r, wf1_r, wb0_r, wb1_r, wr_r, bi_r, bo_r, br_r, o_r):
    invf = 1.0 / jnp.maximum(df_r[...], 1.0)
    invb = 1.0 / jnp.maximum(db_r[...], 1.0)
    fwd = jnp.dot(sf0_r[...] * invf, wf0_r[...],
                  preferred_element_type=jnp.float32)
    fwd = fwd + jnp.dot(sf1_r[...] * invf, wf1_r[...],
                        preferred_element_type=jnp.float32)
    bwd = jnp.dot(sb0_r[...] * invb, wb0_r[...],
                  preferred_element_type=jnp.float32)
    bwd = bwd + jnp.dot(sb1_r[...] * invb, wb1_r[...],
                        preferred_element_type=jnp.float32)
    root = jnp.dot(x_r[...], wr_r[...], preferred_element_type=jnp.float32)
    bias = ALPHA * bi_r[...] + (1.0 - ALPHA) * bo_r[...] + br_r[...]
    o_r[...] = ALPHA * fwd + (1.0 - ALPHA) * bwd + root + bias

  half = pl.BlockSpec((R, H), lambda i: (i, 0))
  col = pl.BlockSpec((R, 1), lambda i: (i, 0))
  whole = lambda s: pl.BlockSpec(s, lambda i: (0, 0))
  return pl.pallas_call(
      body,
      grid=(N_NODES // R,),
      in_specs=[half, half, half, half, col, col,
                pl.BlockSpec((R, D), lambda i: (i, 0)),
                whole((H, D)), whole((H, D)), whole((H, D)), whole((H, D)),
                whole((D, D)), whole((1, D)), whole((1, D)), whole((1, D))],
      out_specs=pl.BlockSpec((R, D), lambda i: (i, 0)),
      out_shape=jax.ShapeDtypeStruct((N_NODES, D), jnp.float32),
  )(sf0, sf1, sb0, sb1, df, db, x,
    wf0, wf1, wb0, wb1, wr, b_in, b_out, b_root)


def kernel(x, edge_index, W_in, b_in, W_out, b_out, W_root, b_root):
  ei = edge_index.astype(jnp.int32)
  src, dst = ei[0], ei[1]
  xr = x.reshape(2 * N_NODES, H)
  gf = jnp.concatenate([2 * src, 2 * src + 1])
  gb = jnp.concatenate([2 * dst, 2 * dst + 1])
  ds_flat = jnp.concatenate([dst, src])
  zeros_blk = jnp.zeros((B, H), jnp.float32)
  ones_blk = jnp.ones((B, H), jnp.float32)

  sf_big, sb_big, deg_big = _sc_segment_sums(
      xr, gf, gb, src, dst, ds_flat, zeros_blk, ones_blk)

  sf0, sf1 = sf_big[:N_NODES], sf_big[N_PAD:N_PAD + N_NODES]
  sb0, sb1 = sb_big[:N_NODES], sb_big[N_PAD:N_PAD + N_NODES]
  df = deg_big[:N_NODES, :1]
  db = deg_big[N_PAD:N_PAD + N_NODES, :1]

  wf = W_in.T
  wb = W_out.T
  return _tc_combine(
      sf0, sf1, sb0, sb1, df, db, x,
      wf[:H], wf[H:], wb[:H], wb[H:], W_root.T,
      b_in.reshape(1, D), b_out.reshape(1, D), b_root.reshape(1, D))


# speedup vs baseline: 3.9632x; 1.1382x over previous
import functools

import jax
import jax.numpy as jnp
from jax import lax
from jax.experimental import pallas as pl
from jax.experimental.pallas import tpu as pltpu
from jax.experimental.pallas import tpu_sc as plsc

N_NODES = 10000
N_EDGES = 160000
D = 256
ALPHA = 0.5

H = 128
NS = 16
EPT = N_EDGES // NS
B = 80
NB = EPT // B
N_PAD = 10240
SLAB = N_PAD // NS
NCH = SLAB // B
DEG_W = 16


def _sc_segment_sums(xr, gf, gb, src, dst, ds_flat, zeros_blk, ones_blk):
  mesh = plsc.VectorSubcoreMesh(core_axis_name="c", subcore_axis_name="s")
  out_type = [
      jax.ShapeDtypeStruct((2 * N_PAD, H), jnp.float32),
      jax.ShapeDtypeStruct((2 * N_PAD, H), jnp.float32),
      jax.ShapeDtypeStruct((2 * N_PAD, H), jnp.float32),
  ]

  @functools.partial(
      pl.kernel,
      out_type=out_type,
      mesh=mesh,
      scratch_types=[
          pltpu.VMEM_SHARED((N_PAD, H), jnp.float32),
          pltpu.VMEM((B,), jnp.int32),
          pltpu.VMEM((B,), jnp.int32),
          pltpu.VMEM((B,), jnp.int32),
          pltpu.VMEM((B,), jnp.int32),
          pltpu.VMEM((B, H), jnp.float32),
          pltpu.VMEM((B, H), jnp.float32),
          pltpu.SemaphoreType.DMA,
          pltpu.SemaphoreType.DMA,
          pltpu.SemaphoreType.DMA,
          pltpu.SemaphoreType.DMA,
          pltpu.SemaphoreType.DMA,
      ],
  )
  def sc_kernel(xr_hbm, gf_hbm, gb_hbm, src_hbm, dst_hbm, ds_hbm, zf_hbm,
                ones_hbm, sf_big, sb_big, deg_big,
                acc, idx_g0, idx_s0, idx_g1, idx_s1, rows0, rows1,
                sem0, sem1, sem2, sem3, semi):
    cid = lax.axis_index("c")
    sid = lax.axis_index("s")
    z0 = sid * SLAB
    e0 = sid * EPT
    g0 = cid * N_EDGES + e0
    o0 = cid * N_PAD + z0

    def zero_acc():
      zcps = [pltpu.async_copy(zf_hbm, acc.at[pl.ds(z0 + k * B, B)], sem2)
              for k in range(NCH)]
      for cp in zcps:
        cp.wait()

    def drain_acc(out_big):
      bufs = [rows0, rows1]
      descs = [None, None]
      for k in range(NCH):
        buf = bufs[k % 2]
        if descs[k % 2] is not None:
          descs[k % 2].wait()
        pltpu.sync_copy(acc.at[pl.ds(z0 + k * B, B)], buf)
        descs[k % 2] = pltpu.async_copy(
            buf, out_big.at[pl.ds(o0 + k * B, B)], sem3)
      descs[NCH % 2].wait()
      descs[(NCH + 1) % 2].wait()
      plsc.subcore_barrier()

    def feat_pass(g_hbm, s_hbm, out_big):
      zero_acc()
      plsc.subcore_barrier()

      def body2(t, carry):
        b0 = 2 * t * B
        b1 = b0 + B
        i0 = pltpu.async_copy(g_hbm.at[pl.ds(g0 + b0, B)], idx_g0, semi)
        i1 = pltpu.async_copy(s_hbm.at[pl.ds(e0 + b0, B)], idx_s0, semi)
        i2 = pltpu.async_copy(g_hbm.at[pl.ds(g0 + b1, B)], idx_g1, semi)
        i3 = pltpu.async_copy(s_hbm.at[pl.ds(e0 + b1, B)], idx_s1, semi)
        i0.wait()
        i1.wait()
        cp0 = pltpu.async_copy(xr_hbm.at[idx_g0], rows0, sem0)
        i2.wait()
        i3.wait()
        cp1 = pltpu.async_copy(xr_hbm.at[idx_g1], rows1, sem1)
        cp0.wait()
        sc0 = pltpu.async_copy(rows0, acc.at[idx_s0], sem2, add=True)
        cp1.wait()
        sc1 = pltpu.async_copy(rows1, acc.at[idx_s1], sem3, add=True)
        sc0.wait()
        sc1.wait()
        return carry

      lax.fori_loop(0, NB // 2, body2, 0)
      bt = (NB - 1) * B
      pltpu.sync_copy(g_hbm.at[pl.ds(g0 + bt, B)], idx_g0)
      pltpu.sync_copy(s_hbm.at[pl.ds(e0 + bt, B)], idx_s0)
      pltpu.async_copy(xr_hbm.at[idx_g0], rows0, sem0).wait()
      pltpu.sync_copy(rows0, acc.at[idx_s0], add=True)
      plsc.subcore_barrier()
      drain_acc(out_big)

    def deg_pass():
      zero_acc()
      pltpu.sync_copy(ones_hbm, rows0)
      plsc.subcore_barrier()

      def body2(t, carry):
        b0 = 2 * t * B
        b1 = b0 + B
        i0 = pltpu.async_copy(ds_hbm.at[pl.ds(g0 + b0, B)], idx_s0, semi)
        i1 = pltpu.async_copy(ds_hbm.at[pl.ds(g0 + b1, B)], idx_s1, semi)
        i0.wait()
        cp0 = pltpu.async_copy(rows0, acc.at[idx_s0], sem0, add=True)
        i1.wait()
        cp1 = pltpu.async_copy(rows0, acc.at[idx_s1], sem1, add=True)
        cp0.wait()
        cp1.wait()
        return carry

      lax.fori_loop(0, NB // 2, body2, 0)
      bt = (NB - 1) * B
      pltpu.sync_copy(ds_hbm.at[pl.ds(g0 + bt, B)], idx_s0)
      pltpu.sync_copy(rows0, acc.at[idx_s0], add=True)
      plsc.subcore_barrier()
      drain_acc(deg_big)

    feat_pass(gf_hbm, dst_hbm, sf_big)
    feat_pass(gb_hbm, src_hbm, sb_big)
    deg_pass()

  return sc_kernel(xr, gf, gb, src, dst, ds_flat, zeros_blk, ones_blk)


R = 400


def _tc_combine(sf0, sf1, sb0, sb1, df, db, x,
                wf0, wf1, wb0, wb1, wr, b_in, b_out, b_root):
  def body(sf0_r, sf1_r, sb0_r, sb1_r, df_r, db_r, x_r,
           wf0_r, wf1_r, wb0_r, wb1_r, wr_r, bi_r, bo_r, br_r, o_r):
    invf = 1.0 / jnp.maximum(df_r[...], 1.0)
    invb = 1.0 / jnp.maximum(db_r[...], 1.0)
    fwd = jnp.dot(sf0_r[...] * invf, wf0_r[...],
                  preferred_element_type=jnp.float32)
    fwd = fwd + jnp.dot(sf1_r[...] * invf, wf1_r[...],
                        preferred_element_type=jnp.float32)
    bwd = jnp.dot(sb0_r[...] * invb, wb0_r[...],
                  preferred_element_type=jnp.float32)
    bwd = bwd + jnp.dot(sb1_r[...] * invb, wb1_r[...],
                        preferred_element_type=jnp.float32)
    root = jnp.dot(x_r[...], wr_r[...], preferred_element_type=jnp.float32)
    bias = ALPHA * bi_r[...] + (1.0 - ALPHA) * bo_r[...] + br_r[...]
    o_r[...] = ALPHA * fwd + (1.0 - ALPHA) * bwd + root + bias

  half = pl.BlockSpec((R, H), lambda i: (i, 0))
  col = pl.BlockSpec((R, 1), lambda i: (i, 0))
  whole = lambda s: pl.BlockSpec(s, lambda i: (0, 0))
  return pl.pallas_call(
      body,
      grid=(N_NODES // R,),
      in_specs=[half, half, half, half, col, col,
                pl.BlockSpec((R, D), lambda i: (i, 0)),
                whole((H, D)), whole((H, D)), whole((H, D)), whole((H, D)),
                whole((D, D)), whole((1, D)), whole((1, D)), whole((1, D))],
      out_specs=pl.BlockSpec((R, D), lambda i: (i, 0)),
      out_shape=jax.ShapeDtypeStruct((N_NODES, D), jnp.float32),
  )(sf0, sf1, sb0, sb1, df, db, x,
    wf0, wf1, wb0, wb1, wr, b_in, b_out, b_root)


def kernel(x, edge_index, W_in, b_in, W_out, b_out, W_root, b_root):
  ei = edge_index.astype(jnp.int32)
  src, dst = ei[0], ei[1]
  xr = x.reshape(2 * N_NODES, H)
  gf = jnp.concatenate([2 * src, 2 * src + 1])
  gb = jnp.concatenate([2 * dst, 2 * dst + 1])
  ds_flat = jnp.concatenate([dst, src])
  zeros_blk = jnp.zeros((B, H), jnp.float32)
  ones_blk = jnp.ones((B, H), jnp.float32)

  sf_big, sb_big, deg_big = _sc_segment_sums(
      xr, gf, gb, src, dst, ds_flat, zeros_blk, ones_blk)

  sf0, sf1 = sf_big[:N_NODES], sf_big[N_PAD:N_PAD + N_NODES]
  sb0, sb1 = sb_big[:N_NODES], sb_big[N_PAD:N_PAD + N_NODES]
  df = deg_big[:N_NODES, :1]
  db = deg_big[N_PAD:N_PAD + N_NODES, :1]

  wf = W_in.T
  wb = W_out.T
  return _tc_combine(
      sf0, sf1, sb0, sb1, df, db, x,
      wf[:H], wf[H:], wb[:H], wb[H:], W_root.T,
      b_in.reshape(1, D), b_out.reshape(1, D), b_root.reshape(1, D))
